# Initial kernel scaffold; baseline (speedup 1.0000x reference)
import jax
import jax.numpy as jnp
from jax.experimental import pallas as pl


def kernel(x, edge_index, W1, b1, W2, b2, cache_name):
    raise NotImplementedError("write your pallas kernel here")


# speedup vs baseline: 22.5295x; 22.5295x over previous
import functools

import jax
import jax.numpy as jnp
from jax import lax
from jax.experimental import pallas as pl
from jax.experimental.pallas import tpu as pltpu
from jax.experimental.pallas import tpu_sc as plsc

N = 10000
E = 320000
D = 128
NC = 2
NS = 16
NW = NC * NS
NPAD = 10240
EPAD = 327680
CH = 128
NCHUNK = EPAD // NW // CH
RPT = NPAD // NS
ZR = 16
GRID = 10
BR = NPAD // GRID

_sc_mesh = plsc.VectorSubcoreMesh(core_axis_name="c", subcore_axis_name="s")


@functools.partial(
    pl.kernel,
    mesh=_sc_mesh,
    out_type=jax.ShapeDtypeStruct((NC * NPAD,), jnp.float32),
    scratch_types=[
        pltpu.VMEM((NCHUNK, CH), jnp.int32),
        pltpu.VMEM((CH,), jnp.float32),
        pltpu.VMEM((CH,), jnp.float32),
        pltpu.VMEM_SHARED((NPAD,), jnp.float32),
    ],
)
def _deg_kernel(dst_hbm, out_hbm, idx_v, ones_v, zeros_v, deg_sh):
    c = lax.axis_index("c")
    s = lax.axis_index("s")
    wid = c * NS + s
    for k in range(CH // 16):
        ones_v[pl.ds(k * 16, 16)] = jnp.ones((16,), jnp.float32)
        zeros_v[pl.ds(k * 16, 16)] = jnp.zeros((16,), jnp.float32)
    base = s * RPT
    for k in range(RPT // CH):
        pltpu.sync_copy(zeros_v, deg_sh.at[pl.ds(base + k * CH, CH)])
    pltpu.sync_copy(dst_hbm.at[pl.ds(wid * NCHUNK, NCHUNK)], idx_v)
    plsc.subcore_barrier()

    def body(j, carry):
        pltpu.sync_copy(ones_v, deg_sh.at[idx_v.at[j]], add=True)
        return carry

    lax.fori_loop(0, NCHUNK, body, 0)
    plsc.subcore_barrier()
    pltpu.sync_copy(deg_sh.at[pl.ds(base, RPT)],
                    out_hbm.at[pl.ds(c * NPAD + base, RPT)])


@functools.partial(
    pl.kernel,
    mesh=_sc_mesh,
    out_type=jax.ShapeDtypeStruct((NC * NPAD, D), jnp.float32),
    scratch_types=[
        pltpu.VMEM((NCHUNK, CH), jnp.int32),
        pltpu.VMEM((NCHUNK, CH), jnp.int32),
        pltpu.VMEM((CH, D), jnp.float32),
        pltpu.VMEM((ZR, D), jnp.float32),
        pltpu.VMEM_SHARED((NPAD, D), jnp.float32),
        pltpu.SemaphoreType.DMA,
    ],
)
def _scatter_kernel(g_hbm, src_hbm, dst_hbm, out_hbm,
                    src_v, dst_v, rows_v, z_v, acc_sh, sem):
    c = lax.axis_index("c")
    s = lax.axis_index("s")
    wid = c * NS + s
    for r in range(ZR):
        for k in range(D // 16):
            z_v[r, pl.ds(k * 16, 16)] = jnp.zeros((16,), jnp.float32)
    rbase = s * RPT
    for k in range(RPT // ZR):
        pltpu.sync_copy(z_v, acc_sh.at[pl.ds(rbase + k * ZR, ZR)])
    pltpu.sync_copy(src_hbm.at[pl.ds(wid * NCHUNK, NCHUNK)], src_v)
    pltpu.sync_copy(dst_hbm.at[pl.ds(wid * NCHUNK, NCHUNK)], dst_v)
    plsc.subcore_barrier()

    def body(j, carry):
        pltpu.async_copy(g_hbm.at[src_v.at[j]], rows_v, sem).wait()
        pltpu.sync_copy(rows_v, acc_sh.at[dst_v.at[j]], add=True)
        return carry

    lax.fori_loop(0, NCHUNK, body, 0)
    plsc.subcore_barrier()
    pltpu.sync_copy(acc_sh.at[pl.ds(rbase, RPT)],
                    out_hbm.at[pl.ds(c * NPAD + rbase, RPT)])


def _tc1_body(d0, d1, x, w, dinv_ref, g_ref):
    deg = d0[...] + d1[...] + 1.0
    dinv = lax.rsqrt(deg)
    dinv_ref[...] = dinv
    g_ref[...] = dinv * jnp.dot(x[...], w[...],
                                preferred_element_type=jnp.float32)


_tc1_call = pl.pallas_call(
    _tc1_body,
    grid=(GRID,),
    in_specs=[
        pl.BlockSpec((BR, 1), lambda i: (i, 0)),
        pl.BlockSpec((BR, 1), lambda i: (i + GRID, 0)),
        pl.BlockSpec((BR, D), lambda i: (i, 0)),
        pl.BlockSpec((D, D), lambda i: (0, 0)),
    ],
    out_specs=[
        pl.BlockSpec((BR, 1), lambda i: (i, 0)),
        pl.BlockSpec((BR, D), lambda i: (i, 0)),
    ],
    out_shape=[
        jax.ShapeDtypeStruct((NPAD, 1), jnp.float32),
        jax.ShapeDtypeStruct((NPAD, D), jnp.float32),
    ],
)


def _tc2_body(p0, p1, g1, dinv, b, w, out_ref):
    t = dinv[...] * (p0[...] + p1[...] + g1[...]) + b[...]
    z = jnp.maximum(t, 0.0)
    out_ref[...] = dinv[...] * jnp.dot(z, w[...],
                                       preferred_element_type=jnp.float32)


_tc2_call = pl.pallas_call(
    _tc2_body,
    grid=(GRID,),
    in_specs=[
        pl.BlockSpec((BR, D), lambda i: (i, 0)),
        pl.BlockSpec((BR, D), lambda i: (i + GRID, 0)),
        pl.BlockSpec((BR, D), lambda i: (i, 0)),
        pl.BlockSpec((BR, 1), lambda i: (i, 0)),
        pl.BlockSpec((1, D), lambda i: (0, 0)),
        pl.BlockSpec((D, D), lambda i: (0, 0)),
    ],
    out_specs=pl.BlockSpec((BR, D), lambda i: (i, 0)),
    out_shape=jax.ShapeDtypeStruct((NPAD, D), jnp.float32),
)


def _tc3_body(q0, q1, g2, dinv, b, out_ref):
    out_ref[...] = dinv[...] * (q0[...] + q1[...] + g2[...]) + b[...]


_tc3_call = pl.pallas_call(
    _tc3_body,
    grid=(GRID,),
    in_specs=[
        pl.BlockSpec((BR, D), lambda i: (i, 0)),
        pl.BlockSpec((BR, D), lambda i: (i + GRID, 0)),
        pl.BlockSpec((BR, D), lambda i: (i, 0)),
        pl.BlockSpec((BR, 1), lambda i: (i, 0)),
        pl.BlockSpec((1, D), lambda i: (0, 0)),
    ],
    out_specs=pl.BlockSpec((BR, D), lambda i: (i, 0)),
    out_shape=jax.ShapeDtypeStruct((N, D), jnp.float32),
)


def kernel(x, edge_index, W1, b1, W2, b2, cache_name):
    src = edge_index[0]
    dst = edge_index[1]
    pad = (N + (jnp.arange(EPAD - E, dtype=jnp.int32) % (NPAD - N))
           ).astype(jnp.int32)
    src_p = jnp.concatenate([src, pad]).reshape(NW * NCHUNK, CH)
    dst_p = jnp.concatenate([dst, pad]).reshape(NW * NCHUNK, CH)

    degs = _deg_kernel(dst_p).reshape(NC * NPAD, 1)
    xp = jnp.pad(x, ((0, NPAD - N), (0, 0)))
    dinv, g1 = _tc1_call(degs, degs, xp, W1)
    p = _scatter_kernel(g1, src_p, dst_p)
    g2 = _tc2_call(p, p, g1, dinv, b1.reshape(1, D), W2)
    q = _scatter_kernel(g2, src_p, dst_p)
    out = _tc3_call(q, q, g2, dinv, b2.reshape(1, D))
    return out


# speedup vs baseline: 32.0076x; 1.4207x over previous
import functools

import jax
import jax.numpy as jnp
from jax import lax
from jax.experimental import pallas as pl
from jax.experimental.pallas import tpu as pltpu
from jax.experimental.pallas import tpu_sc as plsc

N = 10000
E = 320000
D = 128
NC = 2
NS = 16
NW = NC * NS
NPAD = 10240
EPAD = 327680
CH = 128
NCHUNK = EPAD // NW // CH
RPT = NPAD // NS
NBUF = 2
NPH = 2
HCH = 40
GRID = 10
BR = NPAD // GRID

_sc_mesh = plsc.VectorSubcoreMesh(core_axis_name="c", subcore_axis_name="s")


@functools.partial(
    pl.kernel,
    mesh=_sc_mesh,
    out_type=jax.ShapeDtypeStruct((NC * NPAD,), jnp.float32),
    scratch_types=[
        pltpu.VMEM((NCHUNK, CH), jnp.int32),
        pltpu.VMEM((CH,), jnp.float32),
        pltpu.VMEM((CH,), jnp.float32),
        pltpu.VMEM_SHARED((NPAD,), jnp.float32),
    ],
)
def _deg_kernel(dst_hbm, out_hbm, idx_v, ones_v, zeros_v, deg_sh):
    c = lax.axis_index("c")
    s = lax.axis_index("s")
    wid = c * NS + s
    for k in range(CH // 16):
        ones_v[pl.ds(k * 16, 16)] = jnp.ones((16,), jnp.float32)
        zeros_v[pl.ds(k * 16, 16)] = jnp.zeros((16,), jnp.float32)
    base = s * RPT
    for k in range(RPT // CH):
        pltpu.sync_copy(zeros_v, deg_sh.at[pl.ds(base + k * CH, CH)])
    pltpu.sync_copy(dst_hbm.at[pl.ds(wid * NCHUNK, NCHUNK)], idx_v)
    plsc.subcore_barrier()

    def body(j, carry):
        pltpu.sync_copy(ones_v, deg_sh.at[idx_v.at[j]], add=True)
        return carry

    lax.fori_loop(0, NCHUNK, body, 0)
    plsc.subcore_barrier()
    pltpu.sync_copy(deg_sh.at[pl.ds(base, RPT)],
                    out_hbm.at[pl.ds(c * NPAD + base, RPT)])


@functools.partial(
    pl.kernel,
    mesh=_sc_mesh,
    out_type=jax.ShapeDtypeStruct((NC * NPAD, D), jnp.float32),
    scratch_types=[
        pltpu.VMEM((HCH, CH), jnp.int32),
        pltpu.VMEM((HCH, CH), jnp.int32),
        pltpu.VMEM((NBUF, CH, D), jnp.float32),
        pltpu.VMEM_SHARED((NPAD, D), jnp.float32),
        pltpu.SemaphoreType.DMA,
        pltpu.SemaphoreType.DMA,
    ],
)
def _scatter_kernel(g_hbm, src_hbm, dst_hbm, out_hbm,
                    src_v, dst_v, rows_v, acc_sh, sem0, sem1):
    sems = (sem0, sem1)
    c = lax.axis_index("c")
    s = lax.axis_index("s")
    wid = c * NS + s
    rbase = s * RPT
    pltpu.sync_copy(g_hbm.at[pl.ds(rbase, RPT)], acc_sh.at[pl.ds(rbase, RPT)])
    plsc.subcore_barrier()

    for p in range(NPH):
        pltpu.sync_copy(src_hbm.at[pl.ds(wid * NCHUNK + p * HCH, HCH)], src_v)
        pltpu.sync_copy(dst_hbm.at[pl.ds(wid * NCHUNK + p * HCH, HCH)], dst_v)
        for b in range(NBUF):
            pltpu.async_copy(g_hbm.at[src_v.at[b]], rows_v.at[b], sems[b])

        def outer(i, carry):
            j0 = i * NBUF
            for b in range(NBUF):
                j = j0 + b
                pltpu.make_async_copy(g_hbm.at[src_v.at[j]], rows_v.at[b],
                                      sems[b]).wait()
                pltpu.sync_copy(rows_v.at[b], acc_sh.at[dst_v.at[j]],
                                add=True)
                pltpu.async_copy(g_hbm.at[src_v.at[j + NBUF]], rows_v.at[b],
                                 sems[b])
            return carry

        lax.fori_loop(0, HCH // NBUF - 1, outer, 0)
        for b in range(NBUF):
            j = HCH - NBUF + b
            pltpu.make_async_copy(g_hbm.at[src_v.at[j]], rows_v.at[b],
                                  sems[b]).wait()
            pltpu.sync_copy(rows_v.at[b], acc_sh.at[dst_v.at[j]], add=True)
    plsc.subcore_barrier()
    pltpu.sync_copy(acc_sh.at[pl.ds(rbase, RPT)],
                    out_hbm.at[pl.ds(c * NPAD + rbase, RPT)])


def _tc1_body(d0, d1, x, w, dinv_ref, g_ref):
    deg = d0[...] + d1[...] + 1.0
    dinv = lax.rsqrt(deg)
    dinv_ref[...] = dinv
    g_ref[...] = dinv * jnp.dot(x[...], w[...],
                                preferred_element_type=jnp.float32)


_tc1_call = pl.pallas_call(
    _tc1_body,
    grid=(GRID,),
    in_specs=[
        pl.BlockSpec((BR, 1), lambda i: (i, 0)),
        pl.BlockSpec((BR, 1), lambda i: (i + GRID, 0)),
        pl.BlockSpec((BR, D), lambda i: (i, 0)),
        pl.BlockSpec((D, D), lambda i: (0, 0)),
    ],
    out_specs=[
        pl.BlockSpec((BR, 1), lambda i: (i, 0)),
        pl.BlockSpec((BR, D), lambda i: (i, 0)),
    ],
    out_shape=[
        jax.ShapeDtypeStruct((NPAD, 1), jnp.float32),
        jax.ShapeDtypeStruct((NPAD, D), jnp.float32),
    ],
)


def _tc2_body(p0, p1, g1, dinv, b, w, out_ref):
    t = dinv[...] * (p0[...] + p1[...] - g1[...]) + b[...]
    z = jnp.maximum(t, 0.0)
    out_ref[...] = dinv[...] * jnp.dot(z, w[...],
                                       preferred_element_type=jnp.float32)


_tc2_call = pl.pallas_call(
    _tc2_body,
    grid=(GRID,),
    in_specs=[
        pl.BlockSpec((BR, D), lambda i: (i, 0)),
        pl.BlockSpec((BR, D), lambda i: (i + GRID, 0)),
        pl.BlockSpec((BR, D), lambda i: (i, 0)),
        pl.BlockSpec((BR, 1), lambda i: (i, 0)),
        pl.BlockSpec((1, D), lambda i: (0, 0)),
        pl.BlockSpec((D, D), lambda i: (0, 0)),
    ],
    out_specs=pl.BlockSpec((BR, D), lambda i: (i, 0)),
    out_shape=jax.ShapeDtypeStruct((NPAD, D), jnp.float32),
)


def _tc3_body(q0, q1, g2, dinv, b, out_ref):
    out_ref[...] = dinv[...] * (q0[...] + q1[...] - g2[...]) + b[...]


_tc3_call = pl.pallas_call(
    _tc3_body,
    grid=(GRID,),
    in_specs=[
        pl.BlockSpec((BR, D), lambda i: (i, 0)),
        pl.BlockSpec((BR, D), lambda i: (i + GRID, 0)),
        pl.BlockSpec((BR, D), lambda i: (i, 0)),
        pl.BlockSpec((BR, 1), lambda i: (i, 0)),
        pl.BlockSpec((1, D), lambda i: (0, 0)),
    ],
    out_specs=pl.BlockSpec((BR, D), lambda i: (i, 0)),
    out_shape=jax.ShapeDtypeStruct((N, D), jnp.float32),
)


def kernel(x, edge_index, W1, b1, W2, b2, cache_name):
    src = edge_index[0]
    dst = edge_index[1]
    pad = (N + (jnp.arange(EPAD - E, dtype=jnp.int32) % (NPAD - N))
           ).astype(jnp.int32)
    src_p = jnp.concatenate([src, pad]).reshape(NW * NCHUNK, CH)
    dst_p = jnp.concatenate([dst, pad]).reshape(NW * NCHUNK, CH)

    degs = _deg_kernel(dst_p).reshape(NC * NPAD, 1)
    xp = jnp.pad(x, ((0, NPAD - N), (0, 0)))
    dinv, g1 = _tc1_call(degs, degs, xp, W1)
    p = _scatter_kernel(g1, src_p, dst_p)
    g2 = _tc2_call(p, p, g1, dinv, b1.reshape(1, D), W2)
    q = _scatter_kernel(g2, src_p, dst_p)
    out = _tc3_call(q, q, g2, dinv, b2.reshape(1, D))
    return out


# speedup vs baseline: 32.6758x; 1.0209x over previous
import functools

import jax
import jax.numpy as jnp
from jax import lax
from jax.experimental import pallas as pl
from jax.experimental.pallas import tpu as pltpu
from jax.experimental.pallas import tpu_sc as plsc

N = 10000
E = 320000
D = 128
NC = 2
NS = 16
NW = NC * NS
NPAD = 10240
EPAD = 327680
CH = 128
NCHUNK = EPAD // NW // CH
RPT = NPAD // NS
NBUF = 2
NPH = 2
HCH = 40
GRID = 10
BR = NPAD // GRID

_sc_mesh = plsc.VectorSubcoreMesh(core_axis_name="c", subcore_axis_name="s")


@functools.partial(
    pl.kernel,
    mesh=_sc_mesh,
    out_type=jax.ShapeDtypeStruct((NC * NPAD,), jnp.float32),
    scratch_types=[
        pltpu.VMEM((NCHUNK, CH), jnp.int32),
        pltpu.VMEM((CH,), jnp.float32),
        pltpu.VMEM((CH,), jnp.float32),
        pltpu.VMEM_SHARED((NPAD,), jnp.float32),
        pltpu.SemaphoreType.DMA,
    ],
)
def _deg_kernel(dst_hbm, out_hbm, idx_v, ones_v, zeros_v, deg_sh, dsem):
    c = lax.axis_index("c")
    s = lax.axis_index("s")
    wid = c * NS + s
    for k in range(CH // 16):
        ones_v[pl.ds(k * 16, 16)] = jnp.ones((16,), jnp.float32)
        zeros_v[pl.ds(k * 16, 16)] = jnp.zeros((16,), jnp.float32)
    base = s * RPT
    for k in range(RPT // CH):
        pltpu.sync_copy(zeros_v, deg_sh.at[pl.ds(base + k * CH, CH)])
    pltpu.sync_copy(dst_hbm.at[pl.ds(wid * NCHUNK, NCHUNK)], idx_v)
    plsc.subcore_barrier()

    def body(j, carry):
        pltpu.async_copy(ones_v, deg_sh.at[idx_v.at[j]], dsem, add=True)
        return carry

    lax.fori_loop(0, NCHUNK, body, 0)

    def drain(j, carry):
        pltpu.make_async_copy(ones_v, deg_sh.at[idx_v.at[j]], dsem).wait()
        return carry

    lax.fori_loop(0, NCHUNK, drain, 0)
    plsc.subcore_barrier()
    pltpu.sync_copy(deg_sh.at[pl.ds(base, RPT)],
                    out_hbm.at[pl.ds(c * NPAD + base, RPT)])


@functools.partial(
    pl.kernel,
    mesh=_sc_mesh,
    out_type=jax.ShapeDtypeStruct((NC * NPAD, D), jnp.float32),
    scratch_types=[
        pltpu.VMEM((HCH, CH), jnp.int32),
        pltpu.VMEM((HCH, CH), jnp.int32),
        pltpu.VMEM((NBUF, CH, D), jnp.float32),
        pltpu.VMEM_SHARED((NPAD, D), jnp.float32),
        pltpu.SemaphoreType.DMA,
        pltpu.SemaphoreType.DMA,
    ],
)
def _scatter_kernel(g_hbm, src_hbm, dst_hbm, out_hbm,
                    src_v, dst_v, rows_v, acc_sh, sem0, sem1):
    sems = (sem0, sem1)
    c = lax.axis_index("c")
    s = lax.axis_index("s")
    wid = c * NS + s
    rbase = s * RPT
    pltpu.sync_copy(g_hbm.at[pl.ds(rbase, RPT)], acc_sh.at[pl.ds(rbase, RPT)])
    plsc.subcore_barrier()

    def _gather(j, b):
        for h in range(2):
            pltpu.async_copy(g_hbm.at[src_v.at[j, pl.ds(h * 64, 64)]],
                             rows_v.at[b, pl.ds(h * 64, 64)], sems[b])

    def _wait_gather(j, b):
        for h in range(2):
            pltpu.make_async_copy(g_hbm.at[src_v.at[j, pl.ds(h * 64, 64)]],
                                  rows_v.at[b, pl.ds(h * 64, 64)],
                                  sems[b]).wait()

    for p in range(NPH):
        pltpu.sync_copy(src_hbm.at[pl.ds(wid * NCHUNK + p * HCH, HCH)], src_v)
        pltpu.sync_copy(dst_hbm.at[pl.ds(wid * NCHUNK + p * HCH, HCH)], dst_v)
        for b in range(NBUF):
            _gather(b, b)

        def outer(i, carry):
            j0 = i * NBUF
            for b in range(NBUF):
                j = j0 + b
                _wait_gather(j, b)
                pltpu.sync_copy(rows_v.at[b], acc_sh.at[dst_v.at[j]],
                                add=True)
                _gather(j + NBUF, b)
            return carry

        lax.fori_loop(0, HCH // NBUF - 1, outer, 0)
        for b in range(NBUF):
            j = HCH - NBUF + b
            _wait_gather(j, b)
            pltpu.sync_copy(rows_v.at[b], acc_sh.at[dst_v.at[j]], add=True)
    plsc.subcore_barrier()
    pltpu.sync_copy(acc_sh.at[pl.ds(rbase, RPT)],
                    out_hbm.at[pl.ds(c * NPAD + rbase, RPT)])


def _tc1_body(d0, d1, x, w, dinv_ref, g_ref):
    deg = d0[...] + d1[...] + 1.0
    dinv = lax.rsqrt(deg)
    dinv_ref[...] = dinv
    g_ref[...] = dinv * jnp.dot(x[...], w[...],
                                preferred_element_type=jnp.float32)


_tc1_call = pl.pallas_call(
    _tc1_body,
    grid=(GRID,),
    in_specs=[
        pl.BlockSpec((BR, 1), lambda i: (i, 0)),
        pl.BlockSpec((BR, 1), lambda i: (i + GRID, 0)),
        pl.BlockSpec((BR, D), lambda i: (i, 0)),
        pl.BlockSpec((D, D), lambda i: (0, 0)),
    ],
    out_specs=[
        pl.BlockSpec((BR, 1), lambda i: (i, 0)),
        pl.BlockSpec((BR, D), lambda i: (i, 0)),
    ],
    out_shape=[
        jax.ShapeDtypeStruct((NPAD, 1), jnp.float32),
        jax.ShapeDtypeStruct((NPAD, D), jnp.float32),
    ],
)


def _tc2_body(p0, p1, g1, dinv, b, w, out_ref):
    t = dinv[...] * (p0[...] + p1[...] - g1[...]) + b[...]
    z = jnp.maximum(t, 0.0)
    out_ref[...] = dinv[...] * jnp.dot(z, w[...],
                                       preferred_element_type=jnp.float32)


_tc2_call = pl.pallas_call(
    _tc2_body,
    grid=(GRID,),
    in_specs=[
        pl.BlockSpec((BR, D), lambda i: (i, 0)),
        pl.BlockSpec((BR, D), lambda i: (i + GRID, 0)),
        pl.BlockSpec((BR, D), lambda i: (i, 0)),
        pl.BlockSpec((BR, 1), lambda i: (i, 0)),
        pl.BlockSpec((1, D), lambda i: (0, 0)),
        pl.BlockSpec((D, D), lambda i: (0, 0)),
    ],
    out_specs=pl.BlockSpec((BR, D), lambda i: (i, 0)),
    out_shape=jax.ShapeDtypeStruct((NPAD, D), jnp.float32),
)


def _tc3_body(q0, q1, g2, dinv, b, out_ref):
    out_ref[...] = dinv[...] * (q0[...] + q1[...] - g2[...]) + b[...]


_tc3_call = pl.pallas_call(
    _tc3_body,
    grid=(GRID,),
    in_specs=[
        pl.BlockSpec((BR, D), lambda i: (i, 0)),
        pl.BlockSpec((BR, D), lambda i: (i + GRID, 0)),
        pl.BlockSpec((BR, D), lambda i: (i, 0)),
        pl.BlockSpec((BR, 1), lambda i: (i, 0)),
        pl.BlockSpec((1, D), lambda i: (0, 0)),
    ],
    out_specs=pl.BlockSpec((BR, D), lambda i: (i, 0)),
    out_shape=jax.ShapeDtypeStruct((N, D), jnp.float32),
)


def kernel(x, edge_index, W1, b1, W2, b2, cache_name):
    src = edge_index[0]
    dst = edge_index[1]
    pad = (N + (jnp.arange(EPAD - E, dtype=jnp.int32) % (NPAD - N))
           ).astype(jnp.int32)
    src_p = jnp.concatenate([src, pad]).reshape(NW * NCHUNK, CH)
    dst_p = jnp.concatenate([dst, pad]).reshape(NW * NCHUNK, CH)

    degs = _deg_kernel(dst_p).reshape(NC * NPAD, 1)
    dinv, g1 = _tc1_call(degs, degs, x, W1)
    p = _scatter_kernel(g1, src_p, dst_p)
    g2 = _tc2_call(p, p, g1, dinv, b1.reshape(1, D), W2)
    q = _scatter_kernel(g2, src_p, dst_p)
    out = _tc3_call(q, q, g2, dinv, b2.reshape(1, D))
    return out


# speedup vs baseline: 33.2992x; 1.0191x over previous
import functools

import jax
import jax.numpy as jnp
from jax import lax
from jax.experimental import pallas as pl
from jax.experimental.pallas import tpu as pltpu
from jax.experimental.pallas import tpu_sc as plsc

N = 10000
E = 320000
D = 128
NC = 2
NS = 16
NW = NC * NS
NPAD = 10240
EPAD = 327680
CH = 128
NCHUNK = EPAD // NW // CH
RPT = NPAD // NS
NBUF = 2
NPH = 2
HCH = 40
GRID = 5
BR = NPAD // GRID

_sc_mesh = plsc.VectorSubcoreMesh(core_axis_name="c", subcore_axis_name="s")


@functools.partial(
    pl.kernel,
    mesh=_sc_mesh,
    out_type=jax.ShapeDtypeStruct((NC * NPAD,), jnp.float32),
    scratch_types=[
        pltpu.VMEM((NCHUNK, CH), jnp.int32),
        pltpu.VMEM((CH,), jnp.float32),
        pltpu.VMEM((CH,), jnp.float32),
        pltpu.VMEM_SHARED((NPAD,), jnp.float32),
        pltpu.SemaphoreType.DMA,
    ],
)
def _deg_kernel(dst_hbm, out_hbm, idx_v, ones_v, zeros_v, deg_sh, dsem):
    c = lax.axis_index("c")
    s = lax.axis_index("s")
    wid = c * NS + s
    for k in range(CH // 16):
        ones_v[pl.ds(k * 16, 16)] = jnp.ones((16,), jnp.float32)
        zeros_v[pl.ds(k * 16, 16)] = jnp.zeros((16,), jnp.float32)
    base = s * RPT
    for k in range(RPT // CH):
        pltpu.sync_copy(zeros_v, deg_sh.at[pl.ds(base + k * CH, CH)])
    pltpu.sync_copy(dst_hbm.at[pl.ds(wid * NCHUNK, NCHUNK)], idx_v)
    plsc.subcore_barrier()

    def body(j, carry):
        pltpu.async_copy(ones_v, deg_sh.at[idx_v.at[j]], dsem, add=True)
        return carry

    lax.fori_loop(0, NCHUNK, body, 0)

    def drain(j, carry):
        pltpu.make_async_copy(ones_v, deg_sh.at[idx_v.at[j]], dsem).wait()
        return carry

    lax.fori_loop(0, NCHUNK, drain, 0)
    plsc.subcore_barrier()
    pltpu.sync_copy(deg_sh.at[pl.ds(base, RPT)],
                    out_hbm.at[pl.ds(c * NPAD + base, RPT)])


@functools.partial(
    pl.kernel,
    mesh=_sc_mesh,
    out_type=jax.ShapeDtypeStruct((NC * NPAD, D), jnp.float32),
    scratch_types=[
        pltpu.VMEM((HCH, CH), jnp.int32),
        pltpu.VMEM((HCH, CH), jnp.int32),
        pltpu.VMEM((NBUF, CH, D), jnp.float32),
        pltpu.VMEM_SHARED((NPAD, D), jnp.float32),
        pltpu.SemaphoreType.DMA,
        pltpu.SemaphoreType.DMA,
    ],
)
def _scatter_kernel(g_hbm, src_hbm, dst_hbm, out_hbm,
                    src_v, dst_v, rows_v, acc_sh, sem0, sem1):
    sems = (sem0, sem1)
    c = lax.axis_index("c")
    s = lax.axis_index("s")
    wid = c * NS + s
    rbase = s * RPT
    pltpu.sync_copy(g_hbm.at[pl.ds(rbase, RPT)], acc_sh.at[pl.ds(rbase, RPT)])
    plsc.subcore_barrier()

    def _gather(j, b):
        for h in range(2):
            pltpu.async_copy(g_hbm.at[src_v.at[j, pl.ds(h * 64, 64)]],
                             rows_v.at[b, pl.ds(h * 64, 64)], sems[b])

    def _wait_gather(j, b):
        for h in range(2):
            pltpu.make_async_copy(g_hbm.at[src_v.at[j, pl.ds(h * 64, 64)]],
                                  rows_v.at[b, pl.ds(h * 64, 64)],
                                  sems[b]).wait()

    for p in range(NPH):
        pltpu.sync_copy(src_hbm.at[pl.ds(wid * NCHUNK + p * HCH, HCH)], src_v)
        pltpu.sync_copy(dst_hbm.at[pl.ds(wid * NCHUNK + p * HCH, HCH)], dst_v)
        for b in range(NBUF):
            _gather(b, b)

        def outer(i, carry):
            j0 = i * NBUF
            for b in range(NBUF):
                j = j0 + b
                _wait_gather(j, b)
                pltpu.sync_copy(rows_v.at[b], acc_sh.at[dst_v.at[j]],
                                add=True)
                _gather(j + NBUF, b)
            return carry

        lax.fori_loop(0, HCH // NBUF - 1, outer, 0)
        for b in range(NBUF):
            j = HCH - NBUF + b
            _wait_gather(j, b)
            pltpu.sync_copy(rows_v.at[b], acc_sh.at[dst_v.at[j]], add=True)
    plsc.subcore_barrier()
    pltpu.sync_copy(acc_sh.at[pl.ds(rbase, RPT)],
                    out_hbm.at[pl.ds(c * NPAD + rbase, RPT)])


def _tc1_body(d0, d1, x, w, dinv_ref, g_ref):
    deg = d0[...] + d1[...] + 1.0
    dinv = lax.rsqrt(deg)
    dinv_ref[...] = dinv
    g_ref[...] = dinv * jnp.dot(x[...], w[...],
                                preferred_element_type=jnp.float32)


_tc1_call = pl.pallas_call(
    _tc1_body,
    grid=(GRID,),
    in_specs=[
        pl.BlockSpec((BR, 1), lambda i: (i, 0)),
        pl.BlockSpec((BR, 1), lambda i: (i + GRID, 0)),
        pl.BlockSpec((BR, D), lambda i: (i, 0)),
        pl.BlockSpec((D, D), lambda i: (0, 0)),
    ],
    out_specs=[
        pl.BlockSpec((BR, 1), lambda i: (i, 0)),
        pl.BlockSpec((BR, D), lambda i: (i, 0)),
    ],
    out_shape=[
        jax.ShapeDtypeStruct((NPAD, 1), jnp.float32),
        jax.ShapeDtypeStruct((NPAD, D), jnp.float32),
    ],
)


def _tc2_body(p0, p1, g1, dinv, b, w, out_ref):
    t = dinv[...] * (p0[...] + p1[...] - g1[...]) + b[...]
    z = jnp.maximum(t, 0.0)
    out_ref[...] = dinv[...] * jnp.dot(z, w[...],
                                       preferred_element_type=jnp.float32)


_tc2_call = pl.pallas_call(
    _tc2_body,
    grid=(GRID,),
    in_specs=[
        pl.BlockSpec((BR, D), lambda i: (i, 0)),
        pl.BlockSpec((BR, D), lambda i: (i + GRID, 0)),
        pl.BlockSpec((BR, D), lambda i: (i, 0)),
        pl.BlockSpec((BR, 1), lambda i: (i, 0)),
        pl.BlockSpec((1, D), lambda i: (0, 0)),
        pl.BlockSpec((D, D), lambda i: (0, 0)),
    ],
    out_specs=pl.BlockSpec((BR, D), lambda i: (i, 0)),
    out_shape=jax.ShapeDtypeStruct((NPAD, D), jnp.float32),
)


def _tc3_body(q0, q1, g2, dinv, b, out_ref):
    out_ref[...] = dinv[...] * (q0[...] + q1[...] - g2[...]) + b[...]


_tc3_call = pl.pallas_call(
    _tc3_body,
    grid=(GRID,),
    in_specs=[
        pl.BlockSpec((BR, D), lambda i: (i, 0)),
        pl.BlockSpec((BR, D), lambda i: (i + GRID, 0)),
        pl.BlockSpec((BR, D), lambda i: (i, 0)),
        pl.BlockSpec((BR, 1), lambda i: (i, 0)),
        pl.BlockSpec((1, D), lambda i: (0, 0)),
    ],
    out_specs=pl.BlockSpec((BR, D), lambda i: (i, 0)),
    out_shape=jax.ShapeDtypeStruct((N, D), jnp.float32),
)


def kernel(x, edge_index, W1, b1, W2, b2, cache_name):
    src = edge_index[0]
    dst = edge_index[1]
    pad2d = (N + (jnp.arange(EPAD - E, dtype=jnp.int32) % (NPAD - N))
             ).astype(jnp.int32).reshape((EPAD - E) // CH, CH)
    src_p = jnp.concatenate([src.reshape(E // CH, CH), pad2d])
    dst_p = jnp.concatenate([dst.reshape(E // CH, CH), pad2d])

    degs = _deg_kernel(dst_p).reshape(NC * NPAD, 1)
    dinv, g1 = _tc1_call(degs, degs, x, W1)
    p = _scatter_kernel(g1, src_p, dst_p)
    g2 = _tc2_call(p, p, g1, dinv, b1.reshape(1, D), W2)
    q = _scatter_kernel(g2, src_p, dst_p)
    out = _tc3_call(q, q, g2, dinv, b2.reshape(1, D))
    return out
